# baseline (speedup 1.0000x reference)
import functools
import numpy as np
import jax
import jax.numpy as jnp
from jax import lax
from jax.experimental import pallas as pl
from jax.experimental.pallas import tpu as pltpu
from jax.experimental.pallas import tpu_sc as plsc

B = 4096
NS = 26
D = 128
NF = 27
NIDX = B * NS
VOCAB = 100000


_NC = 2
_NSUB = 16
_NW = _NC * _NSUB


def _sc_gather_body(per_w, chunk, nchunk,
                    idx_hbm, emb_hbm, out_hbm, idx_v, buf0, buf1, sem0, sem1):
  wid = lax.axis_index("s") * _NC + lax.axis_index("c")
  base = wid * per_w
  pltpu.sync_copy(idx_hbm.at[pl.ds(base, per_w)], idx_v)

  def start(c, buf, sem):
    return pltpu.async_copy(emb_hbm.at[idx_v.at[pl.ds(c * chunk, chunk)]],
                            buf, sem)

  start(0, buf0, sem0)

  def body(c, carry):
    @pl.when(c % 2 == 0)
    def _():
      @pl.when(c + 1 < nchunk)
      def _():
        start(c + 1, buf1, sem1)
      pltpu.make_async_copy(emb_hbm.at[idx_v.at[pl.ds(c * chunk, chunk)]],
                            buf0, sem0).wait()
      pltpu.sync_copy(buf0, out_hbm.at[pl.ds(base + c * chunk, chunk)])

    @pl.when(c % 2 == 1)
    def _():
      @pl.when(c + 1 < nchunk)
      def _():
        start(c + 1, buf0, sem0)
      pltpu.make_async_copy(emb_hbm.at[idx_v.at[pl.ds(c * chunk, chunk)]],
                            buf1, sem1).wait()
      pltpu.sync_copy(buf1, out_hbm.at[pl.ds(base + c * chunk, chunk)])
    return carry

  lax.fori_loop(0, nchunk, body, 0)


def _sc_gather(idx, emb, n_idx, chunk):
  per_w = n_idx // _NW
  assert per_w % chunk == 0 and chunk <= 128 and chunk % 8 == 0
  nchunk = per_w // chunk
  mesh = plsc.VectorSubcoreMesh(core_axis_name="c", subcore_axis_name="s")
  f = pl.kernel(
      functools.partial(_sc_gather_body, per_w, chunk, nchunk),
      mesh=mesh,
      out_type=jax.ShapeDtypeStruct((n_idx, D), jnp.float32),
      scratch_types=[
          pltpu.VMEM((per_w,), jnp.int32),
          pltpu.VMEM((chunk, D), jnp.float32),
          pltpu.VMEM((chunk, D), jnp.float32),
          pltpu.SemaphoreType.DMA,
          pltpu.SemaphoreType.DMA,
      ],
  )
  return f(idx, emb)


_BT = 512


def _tc_body(x_ref, embf_ref, bw0_ref, bb0_ref, bw1_ref, bb1_ref, bw2_ref,
             bb2_ref, t0b_ref, wpair_ref, tb0_ref, tw1_ref, tb1_ref, tw2_ref,
             tb2_ref, tw3_ref, tb3_ref, tw4_ref, tb4_ref, out_ref):
  x = x_ref[...]
  h = jnp.maximum(jnp.dot(x, bw0_ref[...],
                          preferred_element_type=jnp.float32) + bb0_ref[...], 0.0)
  h = jnp.maximum(jnp.dot(h, bw1_ref[...],
                          preferred_element_type=jnp.float32) + bb1_ref[...], 0.0)
  bot = jnp.maximum(jnp.dot(h, bw2_ref[...],
                            preferred_element_type=jnp.float32) + bb2_ref[...], 0.0)

  emb3 = embf_ref[...].reshape(_BT, NS, D)
  feat = jnp.concatenate([bot.reshape(_BT, 1, D), emb3], axis=1)
  xact = lax.dot_general(feat, feat,
                         dimension_numbers=(((2,), (2,)), ((0,), (0,))),
                         preferred_element_type=jnp.float32)

  h = jnp.dot(xact.reshape(_BT, NF * NF), wpair_ref[...],
              preferred_element_type=jnp.float32)
  h = h + jnp.dot(bot, t0b_ref[...], preferred_element_type=jnp.float32)
  h = jnp.maximum(h + tb0_ref[...], 0.0)
  h = jnp.maximum(jnp.dot(h, tw1_ref[...],
                          preferred_element_type=jnp.float32) + tb1_ref[...], 0.0)
  h = jnp.maximum(jnp.dot(h, tw2_ref[...],
                          preferred_element_type=jnp.float32) + tb2_ref[...], 0.0)
  h = jnp.maximum(jnp.dot(h, tw3_ref[...],
                          preferred_element_type=jnp.float32) + tb3_ref[...], 0.0)
  out_ref[...] = jnp.dot(h, tw4_ref[...],
                         preferred_element_type=jnp.float32) + tb4_ref[...]


def _const(shape):
  nd = len(shape)
  return pl.BlockSpec(shape, lambda i: (0,) * nd)


def _tc_forward(x, embf, bw0, bb0, bw1, bb1, bw2, bb2, t0b, wpair, tb0, tw1,
                tb1, tw2, tb2, tw3, tb3, tw4, tb4):
  rows = x.shape[0]
  grid = (rows // _BT,)
  return pl.pallas_call(
      _tc_body,
      grid=grid,
      in_specs=[
          pl.BlockSpec((_BT, 13), lambda i: (i, 0)),
          pl.BlockSpec((_BT * NS, D), lambda i: (i, 0)),
          _const((13, 512)),
          _const((1, 512)),
          _const((512, 256)),
          _const((1, 256)),
          _const((256, 128)),
          _const((1, 128)),
          _const((128, 1024)),
          _const((NF * NF, 1024)),
          _const((1, 1024)),
          _const((1024, 1024)),
          _const((1, 1024)),
          _const((1024, 512)),
          _const((1, 512)),
          _const((512, 256)),
          _const((1, 256)),
          _const((256, 1)),
          _const((1, 1)),
      ],
      out_specs=pl.BlockSpec((_BT, 1), lambda i: (i, 0)),
      out_shape=jax.ShapeDtypeStruct((rows, 1), jnp.float32),
  )(x, embf, bw0, bb0, bw1, bb1, bw2, bb2, t0b, wpair, tb0, tw1, tb1, tw2,
    tb2, tw3, tb3, tw4, tb4)


_NG = 4


def kernel(bot_mlp_input, cat_features, bw0, bb0, bw1, bb1, bw2, bb2, emb,
           tw0, tb0, tw1, tb1, tw2, tb2, tw3, tb3, tw4, tb4):
  offsets = jnp.arange(NS, dtype=jnp.int32) * VOCAB
  idx = (cat_features.astype(jnp.int32) + offsets[None, :]).reshape(-1)

  iu = np.triu_indices(NF)
  pmat = np.zeros((NF, NF), dtype=np.int32)
  pmat[iu] = np.arange(NF * (NF + 1) // 2, dtype=np.int32)
  pmat = pmat + pmat.T - np.diag(np.diag(pmat))
  scale = np.full((NF, NF, 1), 0.5, dtype=np.float32)
  scale[np.arange(NF), np.arange(NF), 0] = 1.0
  t0b = tw0[:D]
  wpair = tw0[D:][pmat.reshape(-1)].reshape(NF, NF, 1024) * scale
  wpair = wpair.reshape(NF * NF, 1024)

  bg = B // _NG
  ng_idx = bg * NS
  chunk = (ng_idx // _NW) // 8 if (ng_idx // _NW) % 128 else 128
  outs = []
  for g in range(_NG):
    idx_g = lax.dynamic_slice_in_dim(idx, g * ng_idx, ng_idx)
    embf_g = _sc_gather(idx_g, emb, ng_idx, chunk)
    x_g = lax.dynamic_slice_in_dim(bot_mlp_input, g * bg, bg)
    outs.append(_tc_forward(x_g, embf_g, bw0, bb0.reshape(1, -1), bw1,
                            bb1.reshape(1, -1), bw2, bb2.reshape(1, -1),
                            t0b, wpair,
                            tb0.reshape(1, -1), tw1, tb1.reshape(1, -1),
                            tw2, tb2.reshape(1, -1), tw3, tb3.reshape(1, -1),
                            tw4, tb4.reshape(1, 1)))
  return jnp.concatenate(outs, axis=0)


# speedup vs baseline: 1.2195x; 1.2195x over previous
import functools
import numpy as np
import jax
import jax.numpy as jnp
from jax import lax
from jax.experimental import pallas as pl
from jax.experimental.pallas import tpu as pltpu
from jax.experimental.pallas import tpu_sc as plsc

B = 4096
NS = 26
D = 128
NF = 27
NIDX = B * NS
VOCAB = 100000


_NC = 2
_NSUB = 16
_NW = _NC * _NSUB


def _sc_gather_body(per_w, chunk, nchunk,
                    idx_hbm, emb_hbm, out_hbm, idx_v, buf0, buf1, sem0, sem1):
  wid = lax.axis_index("s") * _NC + lax.axis_index("c")
  base = wid * per_w
  pltpu.sync_copy(idx_hbm.at[pl.ds(base, per_w)], idx_v)

  def start(c, buf, sem):
    return pltpu.async_copy(emb_hbm.at[idx_v.at[pl.ds(c * chunk, chunk)]],
                            buf, sem)

  start(0, buf0, sem0)

  def body(c, carry):
    @pl.when(c % 2 == 0)
    def _():
      @pl.when(c + 1 < nchunk)
      def _():
        start(c + 1, buf1, sem1)
      pltpu.make_async_copy(emb_hbm.at[idx_v.at[pl.ds(c * chunk, chunk)]],
                            buf0, sem0).wait()
      pltpu.sync_copy(buf0, out_hbm.at[pl.ds(base + c * chunk, chunk)])

    @pl.when(c % 2 == 1)
    def _():
      @pl.when(c + 1 < nchunk)
      def _():
        start(c + 1, buf0, sem0)
      pltpu.make_async_copy(emb_hbm.at[idx_v.at[pl.ds(c * chunk, chunk)]],
                            buf1, sem1).wait()
      pltpu.sync_copy(buf1, out_hbm.at[pl.ds(base + c * chunk, chunk)])
    return carry

  lax.fori_loop(0, nchunk, body, 0)


def _sc_gather(idx, emb, n_idx, chunk):
  per_w = n_idx // _NW
  assert per_w % chunk == 0 and chunk <= 128 and chunk % 8 == 0
  nchunk = per_w // chunk
  mesh = plsc.VectorSubcoreMesh(core_axis_name="c", subcore_axis_name="s")
  f = pl.kernel(
      functools.partial(_sc_gather_body, per_w, chunk, nchunk),
      mesh=mesh,
      out_type=jax.ShapeDtypeStruct((n_idx, D), jnp.float32),
      scratch_types=[
          pltpu.VMEM((per_w,), jnp.int32),
          pltpu.VMEM((chunk, D), jnp.float32),
          pltpu.VMEM((chunk, D), jnp.float32),
          pltpu.SemaphoreType.DMA,
          pltpu.SemaphoreType.DMA,
      ],
  )
  return f(idx, emb)


_BT = 512


def _tc_body(x_ref, embf_ref, bw0_ref, bb0_ref, bw1_ref, bb1_ref, bw2_ref,
             bb2_ref, t0b_ref, wpair_ref, tb0_ref, tw1_ref, tb1_ref, tw2_ref,
             tb2_ref, tw3_ref, tb3_ref, tw4_ref, tb4_ref, out_ref):
  x = x_ref[...]
  h = jnp.maximum(jnp.dot(x, bw0_ref[...],
                          preferred_element_type=jnp.float32) + bb0_ref[...], 0.0)
  h = jnp.maximum(jnp.dot(h, bw1_ref[...],
                          preferred_element_type=jnp.float32) + bb1_ref[...], 0.0)
  bot = jnp.maximum(jnp.dot(h, bw2_ref[...],
                            preferred_element_type=jnp.float32) + bb2_ref[...], 0.0)

  emb3 = embf_ref[...].reshape(_BT, NS, D)
  feat = jnp.concatenate([bot.reshape(_BT, 1, D), emb3], axis=1)
  xact = lax.dot_general(feat, feat,
                         dimension_numbers=(((2,), (2,)), ((0,), (0,))),
                         preferred_element_type=jnp.float32)

  h = jnp.dot(xact.reshape(_BT, NF * NF), wpair_ref[...],
              preferred_element_type=jnp.float32)
  h = h + jnp.dot(bot, t0b_ref[...], preferred_element_type=jnp.float32)
  h = jnp.maximum(h + tb0_ref[...], 0.0)
  h = jnp.maximum(jnp.dot(h, tw1_ref[...],
                          preferred_element_type=jnp.float32) + tb1_ref[...], 0.0)
  h = jnp.maximum(jnp.dot(h, tw2_ref[...],
                          preferred_element_type=jnp.float32) + tb2_ref[...], 0.0)
  h = jnp.maximum(jnp.dot(h, tw3_ref[...],
                          preferred_element_type=jnp.float32) + tb3_ref[...], 0.0)
  out_ref[...] = jnp.dot(h, tw4_ref[...],
                         preferred_element_type=jnp.float32) + tb4_ref[...]


def _const(shape):
  nd = len(shape)
  return pl.BlockSpec(shape, lambda i: (0,) * nd)


def _tc_forward(x, embf, bw0, bb0, bw1, bb1, bw2, bb2, t0b, wpair, tb0, tw1,
                tb1, tw2, tb2, tw3, tb3, tw4, tb4):
  rows = x.shape[0]
  grid = (rows // _BT,)
  return pl.pallas_call(
      _tc_body,
      grid=grid,
      in_specs=[
          pl.BlockSpec((_BT, 13), lambda i: (i, 0)),
          pl.BlockSpec((_BT * NS, D), lambda i: (i, 0)),
          _const((13, 512)),
          _const((1, 512)),
          _const((512, 256)),
          _const((1, 256)),
          _const((256, 128)),
          _const((1, 128)),
          _const((128, 1024)),
          _const((NF * NF, 1024)),
          _const((1, 1024)),
          _const((1024, 1024)),
          _const((1, 1024)),
          _const((1024, 512)),
          _const((1, 512)),
          _const((512, 256)),
          _const((1, 256)),
          _const((256, 1)),
          _const((1, 1)),
      ],
      out_specs=pl.BlockSpec((_BT, 1), lambda i: (i, 0)),
      out_shape=jax.ShapeDtypeStruct((rows, 1), jnp.float32),
  )(x, embf, bw0, bb0, bw1, bb1, bw2, bb2, t0b, wpair, tb0, tw1, tb1, tw2,
    tb2, tw3, tb3, tw4, tb4)


_NG = 2


def kernel(bot_mlp_input, cat_features, bw0, bb0, bw1, bb1, bw2, bb2, emb,
           tw0, tb0, tw1, tb1, tw2, tb2, tw3, tb3, tw4, tb4):
  offsets = jnp.arange(NS, dtype=jnp.int32) * VOCAB
  idx = (cat_features.astype(jnp.int32) + offsets[None, :]).reshape(-1)

  iu = np.triu_indices(NF)
  pmat = np.zeros((NF, NF), dtype=np.int32)
  pmat[iu] = np.arange(NF * (NF + 1) // 2, dtype=np.int32)
  pmat = pmat + pmat.T - np.diag(np.diag(pmat))
  scale = np.full((NF, NF, 1), 0.5, dtype=np.float32)
  scale[np.arange(NF), np.arange(NF), 0] = 1.0
  t0b = tw0[:D]
  wpair = tw0[D:][pmat.reshape(-1)].reshape(NF, NF, 1024) * scale
  wpair = wpair.reshape(NF * NF, 1024)

  bg = B // _NG
  ng_idx = bg * NS
  chunk = (ng_idx // _NW) // 8 if (ng_idx // _NW) % 128 else 128
  outs = []
  for g in range(_NG):
    idx_g = lax.dynamic_slice_in_dim(idx, g * ng_idx, ng_idx)
    embf_g = _sc_gather(idx_g, emb, ng_idx, chunk)
    x_g = lax.dynamic_slice_in_dim(bot_mlp_input, g * bg, bg)
    outs.append(_tc_forward(x_g, embf_g, bw0, bb0.reshape(1, -1), bw1,
                            bb1.reshape(1, -1), bw2, bb2.reshape(1, -1),
                            t0b, wpair,
                            tb0.reshape(1, -1), tw1, tb1.reshape(1, -1),
                            tw2, tb2.reshape(1, -1), tw3, tb3.reshape(1, -1),
                            tw4, tb4.reshape(1, 1)))
  return jnp.concatenate(outs, axis=0)
